# Initial kernel scaffold; baseline (speedup 1.0000x reference)
import jax
import jax.numpy as jnp
from jax.experimental import pallas as pl

B, S = 2, 2048
D_MODEL = 1024
N_FEAT = 32
PROTO_DIM = 256
D_FEMB = 128
D_RH = 256
E = 8
DH = 1024
TB = 512


def _moe_block_kernel(
    hidden_ref, feat_ref, proto_ref,
    whctx_ref, wfctx_ref, wfeat_ref, bfeat_ref,
    wr1_ref, br1_ref, wr2_ref, br2_ref,
    we1_ref, be1_ref, we2_ref, be2_ref,
    delta_ref, gw_ref, gl_ref,
):
    h = hidden_ref[0]
    f = feat_ref[0]
    proto = proto_ref[...]

    ctx_h = jnp.dot(proto, whctx_ref[...], preferred_element_type=jnp.float32)
    ctx_f = jnp.dot(proto, wfctx_ref[...], preferred_element_type=jnp.float32)
    hidden_cond = h + ctx_h
    feat_cond = f + ctx_f

    feat_emb = jnp.maximum(
        jnp.dot(feat_cond, wfeat_ref[...], preferred_element_type=jnp.float32)
        + bfeat_ref[...], 0.0)

    router_in = jnp.concatenate([hidden_cond, feat_cond], axis=-1)
    r_h = jnp.maximum(
        jnp.dot(router_in, wr1_ref[...], preferred_element_type=jnp.float32)
        + br1_ref[...], 0.0)
    logits = jnp.dot(r_h, wr2_ref[...], preferred_element_type=jnp.float32) \
        + br2_ref[...]
    gl_ref[0] = logits

    iota = jax.lax.broadcasted_iota(jnp.int32, logits.shape, 1)
    m1 = jnp.max(logits, axis=-1, keepdims=True)
    idx1 = jnp.min(jnp.where(logits == m1, iota, E), axis=-1, keepdims=True)
    sel1 = iota == idx1
    logits2 = jnp.where(sel1, -jnp.inf, logits)
    m2 = jnp.max(logits2, axis=-1, keepdims=True)
    idx2 = jnp.min(jnp.where(logits2 == m2, iota, E), axis=-1, keepdims=True)
    sel2 = iota == idx2
    e2 = jnp.exp(m2 - m1)
    w1 = 1.0 / (1.0 + e2)
    w2 = e2 * w1
    gate = jnp.where(sel1, w1, 0.0) + jnp.where(sel2, w2, 0.0)
    gw_ref[0] = gate

    hc_bf = hidden_cond.astype(jnp.bfloat16)
    fe_bf = feat_emb.astype(jnp.bfloat16)
    acc = jnp.zeros((TB, D_MODEL), jnp.float32)
    for e in range(E):
        w1e = we1_ref[e]
        h1 = jnp.dot(hc_bf, w1e[:D_MODEL], preferred_element_type=jnp.float32)
        h1 = h1 + jnp.dot(fe_bf, w1e[D_MODEL:], preferred_element_type=jnp.float32)
        h1 = jnp.maximum(h1 + be1_ref[e], 0.0)
        oe = jnp.dot(h1.astype(jnp.bfloat16), we2_ref[e],
                     preferred_element_type=jnp.float32) + be2_ref[e]
        acc = acc + gate[:, e:e + 1] * oe
    delta_ref[0] = acc


@jax.jit
def kernel(hidden, feat, proto_context, W_hctx, W_fctx, W_feat, b_feat,
           W_r1, b_r1, W_r2, b_r2, W_e1, b_e1, W_e2, b_e2):
    grid = (B, S // TB)

    def tok_map(b, s):
        return (b, s, 0)

    def rep2(b, s):
        return (0, 0)

    def rep3(b, s):
        return (0, 0, 0)

    in_specs = [
        pl.BlockSpec((1, TB, D_MODEL), tok_map),
        pl.BlockSpec((1, TB, N_FEAT), tok_map),
        pl.BlockSpec((1, PROTO_DIM), lambda b, s: (b, 0)),
        pl.BlockSpec((PROTO_DIM, D_MODEL), rep2),
        pl.BlockSpec((PROTO_DIM, N_FEAT), rep2),
        pl.BlockSpec((N_FEAT, D_FEMB), rep2),
        pl.BlockSpec((1, D_FEMB), rep2),
        pl.BlockSpec((D_MODEL + N_FEAT, D_RH), rep2),
        pl.BlockSpec((1, D_RH), rep2),
        pl.BlockSpec((D_RH, E), rep2),
        pl.BlockSpec((1, E), rep2),
        pl.BlockSpec((E, D_MODEL + D_FEMB, DH), rep3),
        pl.BlockSpec((E, DH), rep2),
        pl.BlockSpec((E, DH, D_MODEL), rep3),
        pl.BlockSpec((E, D_MODEL), rep2),
    ]
    out_specs = [
        pl.BlockSpec((1, TB, D_MODEL), tok_map),
        pl.BlockSpec((1, TB, E), tok_map),
        pl.BlockSpec((1, TB, E), tok_map),
    ]
    out_shape = [
        jax.ShapeDtypeStruct((B, S, D_MODEL), jnp.float32),
        jax.ShapeDtypeStruct((B, S, E), jnp.float32),
        jax.ShapeDtypeStruct((B, S, E), jnp.float32),
    ]

    delta, gate_weights, gate_logits = pl.pallas_call(
        _moe_block_kernel,
        grid=grid,
        in_specs=in_specs,
        out_specs=out_specs,
        out_shape=out_shape,
    )(
        hidden, feat, proto_context,
        W_hctx, W_fctx, W_feat, b_feat.reshape(1, D_FEMB),
        W_r1, b_r1.reshape(1, D_RH), W_r2, b_r2.reshape(1, E),
        W_e1.astype(jnp.bfloat16), b_e1,
        W_e2.astype(jnp.bfloat16), b_e2,
    )
    return delta, gate_weights, gate_logits


# speedup vs baseline: 1.1831x; 1.1831x over previous
import jax
import jax.numpy as jnp
from jax.experimental import pallas as pl

B, S = 2, 2048
D_MODEL = 1024
N_FEAT = 32
PROTO_DIM = 256
D_FEMB = 128
D_RH = 256
E = 8
DH = 1024
TB = 512


def _moe_block_kernel(
    hidden_ref, feat_ref, proto_ref,
    whctx_ref, wfctx_ref, wfeat_ref, bfeat_ref,
    wr1_ref, br1_ref, wr2_ref, br2_ref,
    we1_ref, be1_ref, we2_ref, be2_ref,
    delta_ref, gw_ref, gl_ref,
):
    h = hidden_ref[0]
    f = feat_ref[0]
    b_idx = pl.program_id(0)
    proto = proto_ref[pl.ds(b_idx, 1), :]

    ctx_h = jnp.dot(proto, whctx_ref[...], preferred_element_type=jnp.float32)
    ctx_f = jnp.dot(proto, wfctx_ref[...], preferred_element_type=jnp.float32)
    hidden_cond = h + ctx_h
    feat_cond = f + ctx_f

    feat_emb = jnp.maximum(
        jnp.dot(feat_cond, wfeat_ref[...], preferred_element_type=jnp.float32)
        + bfeat_ref[...], 0.0)

    router_in = jnp.concatenate([hidden_cond, feat_cond], axis=-1)
    r_h = jnp.maximum(
        jnp.dot(router_in, wr1_ref[...], preferred_element_type=jnp.float32)
        + br1_ref[...], 0.0)
    logits = jnp.dot(r_h, wr2_ref[...], preferred_element_type=jnp.float32) \
        + br2_ref[...]
    gl_ref[0] = logits

    iota = jax.lax.broadcasted_iota(jnp.int32, logits.shape, 1)
    m1 = jnp.max(logits, axis=-1, keepdims=True)
    idx1 = jnp.min(jnp.where(logits == m1, iota, E), axis=-1, keepdims=True)
    sel1 = iota == idx1
    logits2 = jnp.where(sel1, -jnp.inf, logits)
    m2 = jnp.max(logits2, axis=-1, keepdims=True)
    idx2 = jnp.min(jnp.where(logits2 == m2, iota, E), axis=-1, keepdims=True)
    sel2 = iota == idx2
    e2 = jnp.exp(m2 - m1)
    w1 = 1.0 / (1.0 + e2)
    w2 = e2 * w1
    gate = jnp.where(sel1, w1, 0.0) + jnp.where(sel2, w2, 0.0)
    gw_ref[0] = gate

    hc_bf = hidden_cond.astype(jnp.bfloat16)
    fe_bf = feat_emb.astype(jnp.bfloat16)
    acc = jnp.zeros((TB, D_MODEL), jnp.float32)
    for e in range(E):
        w1e = we1_ref[e]
        h1 = jnp.dot(hc_bf, w1e[:D_MODEL], preferred_element_type=jnp.float32)
        h1 = h1 + jnp.dot(fe_bf, w1e[D_MODEL:], preferred_element_type=jnp.float32)
        h1 = jnp.maximum(h1 + be1_ref[e], 0.0)
        oe = jnp.dot(h1.astype(jnp.bfloat16), we2_ref[e],
                     preferred_element_type=jnp.float32) + be2_ref[e]
        acc = acc + gate[:, e:e + 1] * oe
    delta_ref[0] = acc


@jax.jit
def kernel(hidden, feat, proto_context, W_hctx, W_fctx, W_feat, b_feat,
           W_r1, b_r1, W_r2, b_r2, W_e1, b_e1, W_e2, b_e2):
    grid = (B, S // TB)

    def tok_map(b, s):
        return (b, s, 0)

    def rep2(b, s):
        return (0, 0)

    def rep3(b, s):
        return (0, 0, 0)

    in_specs = [
        pl.BlockSpec((1, TB, D_MODEL), tok_map),
        pl.BlockSpec((1, TB, N_FEAT), tok_map),
        pl.BlockSpec((B, PROTO_DIM), rep2),
        pl.BlockSpec((PROTO_DIM, D_MODEL), rep2),
        pl.BlockSpec((PROTO_DIM, N_FEAT), rep2),
        pl.BlockSpec((N_FEAT, D_FEMB), rep2),
        pl.BlockSpec((1, D_FEMB), rep2),
        pl.BlockSpec((D_MODEL + N_FEAT, D_RH), rep2),
        pl.BlockSpec((1, D_RH), rep2),
        pl.BlockSpec((D_RH, E), rep2),
        pl.BlockSpec((1, E), rep2),
        pl.BlockSpec((E, D_MODEL + D_FEMB, DH), rep3),
        pl.BlockSpec((E, DH), rep2),
        pl.BlockSpec((E, DH, D_MODEL), rep3),
        pl.BlockSpec((E, D_MODEL), rep2),
    ]
    out_specs = [
        pl.BlockSpec((1, TB, D_MODEL), tok_map),
        pl.BlockSpec((1, TB, E), tok_map),
        pl.BlockSpec((1, TB, E), tok_map),
    ]
    out_shape = [
        jax.ShapeDtypeStruct((B, S, D_MODEL), jnp.float32),
        jax.ShapeDtypeStruct((B, S, E), jnp.float32),
        jax.ShapeDtypeStruct((B, S, E), jnp.float32),
    ]

    delta, gate_weights, gate_logits = pl.pallas_call(
        _moe_block_kernel,
        grid=grid,
        in_specs=in_specs,
        out_specs=out_specs,
        out_shape=out_shape,
    )(
        hidden, feat, proto_context,
        W_hctx, W_fctx, W_feat, b_feat.reshape(1, D_FEMB),
        W_r1, b_r1.reshape(1, D_RH), W_r2, b_r2.reshape(1, E),
        W_e1.astype(jnp.bfloat16), b_e1,
        W_e2.astype(jnp.bfloat16), b_e2,
    )
    return delta, gate_weights, gate_logits
